# baseline (speedup 1.0000x reference)
import functools

import jax
import jax.numpy as jnp
from jax import lax
from jax.experimental import pallas as pl
from jax.experimental.pallas import tpu as pltpu
from jax.experimental.pallas import tpu_sc as plsc

B, S, H, A, K = 2048, 512, 256, 128, 1024
BB = 512
CHUNK = 256
NBLK = B // BB

NC, NS = 2, 16
NW = NC * NS
BPW = B // NW
KPT = K // NS
LANES = 16

_CONTRACT_MINOR = (((1,), (1,)), ((), ()))


def _encoder_body(x_ref, wf_ref, bf_ref, w0_ref, b0_ref, w1_ref, b1_ref,
                  emb_ref, wact_ref, bact_ref, wval_ref, bval_ref,
                  idx_ref, probs_ref, val_ref, norms_ref, vrow_ref):
    i = pl.program_id(0)

    @pl.when(i == 0)
    def _():
        sq = emb_ref[...] * emb_ref[...]
        norms_ref[...] = lax.dot_general(
            jnp.ones((1, H), jnp.float32), sq, _CONTRACT_MINOR,
            preferred_element_type=jnp.float32)
        vrow_ref[...] = lax.dot_general(
            wval_ref[...], emb_ref[...], _CONTRACT_MINOR,
            preferred_element_type=jnp.float32) + bval_ref[...]
        logits = lax.dot_general(emb_ref[...], wact_ref[...], _CONTRACT_MINOR,
                                 preferred_element_type=jnp.float32) + bact_ref[...]
        m = jnp.max(logits, axis=1, keepdims=True)
        e = jnp.exp(logits - m)
        probs_ref[...] = e / jnp.sum(e, axis=1, keepdims=True)

    for c in range(BB // CHUNK):
        x = x_ref[pl.ds(c * CHUNK, CHUNK), :]
        h = jnp.maximum(
            lax.dot_general(x, wf_ref[...], _CONTRACT_MINOR,
                            preferred_element_type=jnp.float32) + bf_ref[...], 0.0)
        h = jnp.maximum(
            lax.dot_general(h, w0_ref[...], _CONTRACT_MINOR,
                            preferred_element_type=jnp.float32) + b0_ref[...], 0.0)
        h = jnp.maximum(
            lax.dot_general(h, w1_ref[...], _CONTRACT_MINOR,
                            preferred_element_type=jnp.float32) + b1_ref[...], 0.0)
        scores = lax.dot_general(h, emb_ref[...], _CONTRACT_MINOR,
                                 preferred_element_type=jnp.float32)
        d2 = (jnp.sum(h * h, axis=1, keepdims=True) - 2.0 * scores
              + norms_ref[...])
        mins = jnp.min(d2, axis=1, keepdims=True)
        ids = lax.broadcasted_iota(jnp.int32, d2.shape, 1)
        idx = jnp.min(jnp.where(d2 == mins, ids, K), axis=1)
        idx_ref[0, 0, pl.ds(c * CHUNK, CHUNK)] = idx
        vsel = jnp.where(ids == idx[:, None], vrow_ref[...], 0.0)
        val_ref[0, 0, pl.ds(c * CHUNK, CHUNK)] = jnp.sum(vsel, axis=1)


_encoder_call = pl.pallas_call(
    _encoder_body,
    grid=(NBLK,),
    in_specs=[
        pl.BlockSpec((BB, S), lambda i: (i, 0)),
        pl.BlockSpec((H, S), lambda i: (0, 0)),
        pl.BlockSpec((1, H), lambda i: (0, 0)),
        pl.BlockSpec((H, H), lambda i: (0, 0)),
        pl.BlockSpec((1, H), lambda i: (0, 0)),
        pl.BlockSpec((H, H), lambda i: (0, 0)),
        pl.BlockSpec((1, H), lambda i: (0, 0)),
        pl.BlockSpec((K, H), lambda i: (0, 0)),
        pl.BlockSpec((A, H), lambda i: (0, 0)),
        pl.BlockSpec((1, A), lambda i: (0, 0)),
        pl.BlockSpec((1, H), lambda i: (0, 0)),
        pl.BlockSpec((1, 1), lambda i: (0, 0)),
    ],
    out_specs=[
        pl.BlockSpec((1, 1, BB), lambda i: (i, 0, 0)),
        pl.BlockSpec((K, A), lambda i: (0, 0)),
        pl.BlockSpec((1, 1, BB), lambda i: (i, 0, 0)),
    ],
    out_shape=[
        jax.ShapeDtypeStruct((NBLK, 1, BB), jnp.int32),
        jax.ShapeDtypeStruct((K, A), jnp.float32),
        jax.ShapeDtypeStruct((NBLK, 1, BB), jnp.float32),
    ],
    scratch_shapes=[
        pltpu.VMEM((1, K), jnp.float32),
        pltpu.VMEM((1, K), jnp.float32),
    ],
)


@functools.cache
def _make_sc_gather():
    @functools.partial(
        pl.kernel,
        mesh=plsc.VectorSubcoreMesh(core_axis_name="c", subcore_axis_name="s"),
        out_type=jax.ShapeDtypeStruct((B, A), jnp.float32),
        scratch_types=[
            pltpu.VMEM((BPW,), jnp.int32),
            pltpu.VMEM((BPW, A), jnp.float32),
            pltpu.VMEM_SHARED((K, A), jnp.float32),
            pltpu.SemaphoreType.DMA,
            pltpu.SemaphoreType.DMA,
        ],
    )
    def _sc_gather(table_hbm, idx_hbm, ap_hbm,
                   idx_v, rows_v, table_sp, sem, sem2):
        cid = lax.axis_index("c")
        sid = lax.axis_index("s")
        wid = sid * NC + cid
        base = wid * BPW
        stage = pltpu.async_copy(table_hbm.at[pl.ds(sid * KPT, KPT)],
                                 table_sp.at[pl.ds(sid * KPT, KPT)], sem)
        pltpu.async_copy(idx_hbm.at[pl.ds(base, BPW)], idx_v, sem2).wait()
        stage.wait()
        plsc.subcore_barrier()
        pltpu.async_copy(table_sp.at[idx_v], rows_v, sem).wait()
        pltpu.sync_copy(rows_v, ap_hbm.at[pl.ds(base, BPW)])

    return _sc_gather


def kernel(inputs, W_first, b_first, W0, b0, W1, b1, W_act, b_act,
           W_val, b_val, embedding):
    idx3, probs_table, val3 = _encoder_call(
        inputs,
        W_first, b_first.reshape(1, H),
        W0, b0.reshape(1, H),
        W1, b1.reshape(1, H),
        embedding,
        W_act, b_act.reshape(1, A),
        W_val, b_val.reshape(1, 1),
    )
    actions_prob = _make_sc_gather()(probs_table, idx3.reshape(B))
    return (actions_prob, val3.reshape(B, 1))


# speedup vs baseline: 1.0841x; 1.0841x over previous
import functools

import jax
import jax.numpy as jnp
from jax import lax
from jax.experimental import pallas as pl
from jax.experimental.pallas import tpu as pltpu
from jax.experimental.pallas import tpu_sc as plsc

B, S, H, A, K = 2048, 512, 256, 128, 1024
BB = 512
NBLK = B // BB

NC, NS = 2, 16
NW = NC * NS
BPW = B // NW
KPT = K // NS
LANES = 16

_CONTRACT_MINOR = (((1,), (1,)), ((), ()))


def _encoder_body(x_ref, wf_ref, bf_ref, w0_ref, b0_ref, w1_ref, b1_ref,
                  emb_ref, wact_ref, bact_ref, wval_ref, bval_ref,
                  idx_ref, probs_ref, norms_ref):
    i = pl.program_id(0)

    @pl.when(i == 0)
    def _():
        sq = emb_ref[...] * emb_ref[...]
        norms_ref[...] = lax.dot_general(
            jnp.ones((1, H), jnp.float32), sq, _CONTRACT_MINOR,
            preferred_element_type=jnp.float32)
        logits = lax.dot_general(emb_ref[...], wact_ref[...], _CONTRACT_MINOR,
                                 preferred_element_type=jnp.float32) + bact_ref[...]
        m = jnp.max(logits, axis=1, keepdims=True)
        e = jnp.exp(logits - m)
        probs_ref[...] = e / jnp.sum(e, axis=1, keepdims=True)

    h = jnp.maximum(
        lax.dot_general(x_ref[...], wf_ref[...], _CONTRACT_MINOR,
                        preferred_element_type=jnp.float32) + bf_ref[...], 0.0)
    h = jnp.maximum(
        lax.dot_general(h, w0_ref[...], _CONTRACT_MINOR,
                        preferred_element_type=jnp.float32) + b0_ref[...], 0.0)
    h = jnp.maximum(
        lax.dot_general(h, w1_ref[...], _CONTRACT_MINOR,
                        preferred_element_type=jnp.float32) + b1_ref[...], 0.0)
    scores = lax.dot_general(h, emb_ref[...], _CONTRACT_MINOR,
                             preferred_element_type=jnp.float32)
    d2 = (jnp.sum(h * h, axis=1, keepdims=True) - 2.0 * scores
          + norms_ref[...])
    mins = jnp.min(d2, axis=1, keepdims=True)
    ids = lax.broadcasted_iota(jnp.int32, d2.shape, 1)
    idx = jnp.min(jnp.where(d2 == mins, ids, K), axis=1)
    idx_ref[...] = idx.reshape(1, 1, BB)


_encoder_call = pl.pallas_call(
    _encoder_body,
    grid=(NBLK,),
    in_specs=[
        pl.BlockSpec((BB, S), lambda i: (i, 0)),
        pl.BlockSpec((H, S), lambda i: (0, 0)),
        pl.BlockSpec((1, H), lambda i: (0, 0)),
        pl.BlockSpec((H, H), lambda i: (0, 0)),
        pl.BlockSpec((1, H), lambda i: (0, 0)),
        pl.BlockSpec((H, H), lambda i: (0, 0)),
        pl.BlockSpec((1, H), lambda i: (0, 0)),
        pl.BlockSpec((K, H), lambda i: (0, 0)),
        pl.BlockSpec((A, H), lambda i: (0, 0)),
        pl.BlockSpec((1, A), lambda i: (0, 0)),
        pl.BlockSpec((1, H), lambda i: (0, 0)),
        pl.BlockSpec((1, 1), lambda i: (0, 0)),
    ],
    out_specs=[
        pl.BlockSpec((1, 1, BB), lambda i: (i, 0, 0)),
        pl.BlockSpec((K, A), lambda i: (0, 0)),
    ],
    out_shape=[
        jax.ShapeDtypeStruct((NBLK, 1, BB), jnp.int32),
        jax.ShapeDtypeStruct((K, A), jnp.float32),
    ],
    scratch_shapes=[
        pltpu.VMEM((1, K), jnp.float32),
    ],
)


def _value_body(idx_ref, emb_ref, wval_ref, bval_ref, val_ref, vrow_ref):
    i = pl.program_id(0)

    @pl.when(i == 0)
    def _():
        vrow_ref[...] = lax.dot_general(
            wval_ref[...], emb_ref[...], _CONTRACT_MINOR,
            preferred_element_type=jnp.float32) + bval_ref[...]

    idx = idx_ref[...].reshape(BB, 1)
    ids = lax.broadcasted_iota(jnp.int32, (BB, K), 1)
    vsel = jnp.where(ids == idx, vrow_ref[...], 0.0)
    val_ref[...] = jnp.sum(vsel, axis=1).reshape(1, 1, BB)


_value_call = pl.pallas_call(
    _value_body,
    grid=(NBLK,),
    in_specs=[
        pl.BlockSpec((1, 1, BB), lambda i: (i, 0, 0)),
        pl.BlockSpec((K, H), lambda i: (0, 0)),
        pl.BlockSpec((1, H), lambda i: (0, 0)),
        pl.BlockSpec((1, 1), lambda i: (0, 0)),
    ],
    out_specs=[pl.BlockSpec((1, 1, BB), lambda i: (i, 0, 0))],
    out_shape=[jax.ShapeDtypeStruct((NBLK, 1, BB), jnp.float32)],
    scratch_shapes=[pltpu.VMEM((1, K), jnp.float32)],
)


@functools.cache
def _make_sc_gather():
    @functools.partial(
        pl.kernel,
        mesh=plsc.VectorSubcoreMesh(core_axis_name="c", subcore_axis_name="s"),
        out_type=jax.ShapeDtypeStruct((B, A), jnp.float32),
        scratch_types=[
            pltpu.VMEM((BPW,), jnp.int32),
            pltpu.VMEM((BPW, A), jnp.float32),
            pltpu.VMEM_SHARED((K, A), jnp.float32),
            pltpu.SemaphoreType.DMA,
            pltpu.SemaphoreType.DMA,
        ],
    )
    def _sc_gather(table_hbm, idx_hbm, ap_hbm,
                   idx_v, rows_v, table_sp, sem, sem2):
        cid = lax.axis_index("c")
        sid = lax.axis_index("s")
        wid = sid * NC + cid
        base = wid * BPW
        stage = pltpu.async_copy(table_hbm.at[pl.ds(sid * KPT, KPT)],
                                 table_sp.at[pl.ds(sid * KPT, KPT)], sem)
        pltpu.async_copy(idx_hbm.at[pl.ds(base, BPW)], idx_v, sem2).wait()
        stage.wait()
        plsc.subcore_barrier()
        pltpu.async_copy(table_sp.at[idx_v], rows_v, sem).wait()
        pltpu.sync_copy(rows_v, ap_hbm.at[pl.ds(base, BPW)])

    return _sc_gather


def kernel(inputs, W_first, b_first, W0, b0, W1, b1, W_act, b_act,
           W_val, b_val, embedding):
    idx3, probs_table = _encoder_call(
        inputs,
        W_first, b_first.reshape(1, H),
        W0, b0.reshape(1, H),
        W1, b1.reshape(1, H),
        embedding,
        W_act, b_act.reshape(1, A),
        W_val, b_val.reshape(1, 1),
    )
    actions_prob = _make_sc_gather()(probs_table, idx3.reshape(B))
    val3, = _value_call(idx3, embedding, W_val, b_val.reshape(1, 1))
    return (actions_prob, val3.reshape(B, 1))
